# Initial kernel scaffold; baseline (speedup 1.0000x reference)
import jax
import jax.numpy as jnp
from jax.experimental import pallas as pl


def kernel(x, edge_index, W1_l, W1_r, b1, W2_l, W2_r, b2):
    raise NotImplementedError("write your pallas kernel here")


# speedup vs baseline: 4.1007x; 4.1007x over previous
import functools

import jax
import jax.numpy as jnp
from jax import lax
from jax.experimental import pallas as pl
from jax.experimental.pallas import tpu as pltpu
from jax.experimental.pallas import tpu_sc as plsc

N = 10000
E = 320000
D = 128
DH = 64
NC = 2
NS = 16
BATCH = 128
NB_S = 160
EP = NS * NB_S * BATCH
NP = 10240
RPT = NP // NS


def _sc_agg_body(with_count, *refs):
    if with_count:
        (x2_hbm, src_hbm, dst_hbm, acc_out, cnt_out,
         src_v, dst_v, rows_v, ones_v, acc_sh, cnt_sh, sem) = refs
    else:
        (x2_hbm, src_hbm, dst_hbm, acc_out,
         src_v, dst_v, rows_v, acc_sh, sem) = refs
    c = lax.axis_index("c")
    s = lax.axis_index("s")
    row0 = s * RPT

    @pl.loop(0, BATCH)
    def _(i):
        @pl.loop(0, DH, step=16)
        def _(k):
            rows_v[i, pl.ds(k, 16)] = jnp.zeros((16,), jnp.float32)

    @pl.loop(0, RPT, step=BATCH)
    def _(r):
        pltpu.sync_copy(rows_v, acc_sh.at[pl.ds(row0 + r, BATCH)])

    if with_count:
        @pl.when(c == 0)
        def _():
            @pl.loop(0, BATCH)
            def _(i):
                ones_v[i, :] = jnp.zeros((16,), jnp.float32)

            @pl.loop(0, RPT, step=BATCH)
            def _(r):
                pltpu.sync_copy(ones_v, cnt_sh.at[pl.ds(row0 + r, BATCH)])

            @pl.loop(0, BATCH)
            def _(i):
                ones_v[i, :] = jnp.ones((16,), jnp.float32)

    pltpu.sync_copy(src_hbm.at[pl.ds(s * NB_S, NB_S)], src_v)
    pltpu.sync_copy(dst_hbm.at[pl.ds(s * NB_S, NB_S)], dst_v)

    plsc.subcore_barrier()

    @pl.loop(0, NB_S)
    def _(j):
        pltpu.async_copy(x2_hbm.at[c].at[src_v.at[j]], rows_v, sem).wait()
        pltpu.sync_copy(rows_v, acc_sh.at[dst_v.at[j]], add=True)
        if with_count:
            @pl.when(c == 0)
            def _():
                pltpu.sync_copy(ones_v, cnt_sh.at[dst_v.at[j]], add=True)

    plsc.subcore_barrier()

    pltpu.sync_copy(acc_sh.at[pl.ds(row0, RPT)], acc_out.at[c, pl.ds(row0, RPT)])
    if with_count:
        @pl.when(c == 0)
        def _():
            pltpu.sync_copy(cnt_sh.at[pl.ds(row0, RPT)],
                            cnt_out.at[pl.ds(row0, RPT)])


def _make_sc_agg(with_count):
    mesh = plsc.VectorSubcoreMesh(core_axis_name="c", subcore_axis_name="s")
    out_type = [jax.ShapeDtypeStruct((NC, NP, DH), jnp.float32)]
    scratch = [
        pltpu.VMEM((NB_S, BATCH), jnp.int32),
        pltpu.VMEM((NB_S, BATCH), jnp.int32),
        pltpu.VMEM((BATCH, DH), jnp.float32),
    ]
    if with_count:
        out_type.append(jax.ShapeDtypeStruct((NP, 16), jnp.float32))
        scratch.append(pltpu.VMEM((BATCH, 16), jnp.float32))
    scratch.append(pltpu.VMEM_SHARED((NP, DH), jnp.float32))
    if with_count:
        scratch.append(pltpu.VMEM_SHARED((NP, 16), jnp.float32))
    scratch.append(pltpu.SemaphoreType.DMA)
    return pl.kernel(
        functools.partial(_sc_agg_body, with_count),
        out_type=out_type,
        mesh=mesh,
        scratch_types=scratch,
        compiler_params=pltpu.CompilerParams(use_tc_tiling_on_sc=False),
    )


def _tc_layer_body(relu, acc_ref, cnt_ref, x_ref, wl_ref, wr_ref, b_ref, o_ref):
    agg = jnp.concatenate([acc_ref[0, :N, :], acc_ref[1, :N, :]], axis=1)
    if x_ref.shape[0] == NC:
        xin = jnp.concatenate([x_ref[0], x_ref[1]], axis=1)
    else:
        xin = x_ref[...]
    cnt = cnt_ref[:N, 0:1]
    mean = agg / jnp.maximum(cnt, 1.0)
    r = (jnp.dot(mean, wl_ref[...], preferred_element_type=jnp.float32)
         + jnp.dot(xin, wr_ref[...], preferred_element_type=jnp.float32)
         + b_ref[...])
    r = jnp.maximum(r, 0.0) if relu else r
    if o_ref.shape[0] == NC:
        o_ref[0] = r[:, :DH]
        o_ref[1] = r[:, DH:]
    else:
        o_ref[...] = r


def _make_tc_layer(relu, split_in, split_out):
    out_shape = (jax.ShapeDtypeStruct((NC, N, DH), jnp.float32) if split_out
                 else jax.ShapeDtypeStruct((N, D), jnp.float32))
    return pl.pallas_call(
        functools.partial(_tc_layer_body, relu),
        out_shape=out_shape,
    )


_sc_agg_cnt = _make_sc_agg(True)
_sc_agg = _make_sc_agg(False)
_tc_layer1 = _make_tc_layer(True, False, True)
_tc_layer2 = _make_tc_layer(False, True, False)


def kernel(x, edge_index, W1_l, W1_r, b1, W2_l, W2_r, b2):
    src = edge_index[0].astype(jnp.int32)
    dst = edge_index[1].astype(jnp.int32)
    pad = EP - E
    src_p = jnp.concatenate([src, jnp.zeros((pad,), jnp.int32)])
    dst_p = jnp.concatenate(
        [dst, N + (jnp.arange(pad, dtype=jnp.int32) % (NP - N))])
    src_p = src_p.reshape(EP // BATCH, BATCH)
    dst_p = dst_p.reshape(EP // BATCH, BATCH)
    x2 = x.reshape(N, NC, DH).transpose(1, 0, 2)

    acc1, cnt1 = _sc_agg_cnt(x2, src_p, dst_p)
    h2 = _tc_layer1(acc1, cnt1, x, W1_l, W1_r, b1.reshape(1, D))
    (acc2,) = _sc_agg(h2, src_p, dst_p)
    out = _tc_layer2(acc2, cnt1, h2, W2_l, W2_r, b2.reshape(1, D))
    return out


# speedup vs baseline: 5.3353x; 1.3011x over previous
import functools

import jax
import jax.numpy as jnp
from jax import lax
from jax.experimental import pallas as pl
from jax.experimental.pallas import tpu as pltpu
from jax.experimental.pallas import tpu_sc as plsc

N = 10000
E = 320000
D = 128
DH = 64
NC = 2
NS = 16
BATCH = 128
NB_S = 160
EP = NS * NB_S * BATCH
NP = 10240
RPT = NP // NS


def _sc_agg_body(with_count, *refs):
    if with_count:
        (x2_hbm, src_hbm, dst_hbm, acc_out, cnt_out,
         src_v, dst_v, rows_v0, rows_v1, ones_v, acc_sh, cnt_sh,
         sem0, sem1) = refs
    else:
        (x2_hbm, src_hbm, dst_hbm, acc_out,
         src_v, dst_v, rows_v0, rows_v1, acc_sh, sem0, sem1) = refs
    c = lax.axis_index("c")
    s = lax.axis_index("s")
    row0 = s * RPT

    rows_v = rows_v0

    @pl.loop(0, BATCH)
    def _(i):
        @pl.loop(0, DH, step=16)
        def _(k):
            rows_v[i, pl.ds(k, 16)] = jnp.zeros((16,), jnp.float32)

    @pl.loop(0, RPT, step=BATCH)
    def _(r):
        pltpu.sync_copy(rows_v, acc_sh.at[pl.ds(row0 + r, BATCH)])

    if with_count:
        @pl.when(c == 0)
        def _():
            @pl.loop(0, BATCH)
            def _(i):
                ones_v[i, :] = jnp.zeros((16,), jnp.float32)

            @pl.loop(0, RPT, step=BATCH)
            def _(r):
                pltpu.sync_copy(ones_v, cnt_sh.at[pl.ds(row0 + r, BATCH)])

            @pl.loop(0, BATCH)
            def _(i):
                ones_v[i, :] = jnp.ones((16,), jnp.float32)

    pltpu.sync_copy(src_hbm.at[pl.ds(s * NB_S, NB_S)], src_v)
    pltpu.sync_copy(dst_hbm.at[pl.ds(s * NB_S, NB_S)], dst_v)

    plsc.subcore_barrier()

    def _gather(j, buf, sem):
        pltpu.async_copy(x2_hbm.at[c].at[src_v.at[j]], buf, sem)

    def _wait(buf, sem):
        pltpu.make_async_copy(x2_hbm.at[c].at[src_v.at[0]], buf, sem).wait()

    def _scat(j, buf):
        pltpu.sync_copy(buf, acc_sh.at[dst_v.at[j]], add=True)
        if with_count:
            @pl.when(c == 0)
            def _():
                pltpu.sync_copy(ones_v, cnt_sh.at[dst_v.at[j]], add=True)

    _gather(0, rows_v0, sem0)

    @pl.loop(0, NB_S - 2, step=2)
    def _(j):
        _gather(j + 1, rows_v1, sem1)
        _wait(rows_v0, sem0)
        _scat(j, rows_v0)
        _gather(j + 2, rows_v0, sem0)
        _wait(rows_v1, sem1)
        _scat(j + 1, rows_v1)

    _gather(NB_S - 1, rows_v1, sem1)
    _wait(rows_v0, sem0)
    _scat(NB_S - 2, rows_v0)
    _wait(rows_v1, sem1)
    _scat(NB_S - 1, rows_v1)

    plsc.subcore_barrier()

    pltpu.sync_copy(acc_sh.at[pl.ds(row0, RPT)], acc_out.at[c, pl.ds(row0, RPT)])
    if with_count:
        @pl.when(c == 0)
        def _():
            pltpu.sync_copy(cnt_sh.at[pl.ds(row0, RPT)],
                            cnt_out.at[pl.ds(row0, RPT)])


def _make_sc_agg(with_count):
    mesh = plsc.VectorSubcoreMesh(core_axis_name="c", subcore_axis_name="s")
    out_type = [jax.ShapeDtypeStruct((NC, NP, DH), jnp.float32)]
    scratch = [
        pltpu.VMEM((NB_S, BATCH), jnp.int32),
        pltpu.VMEM((NB_S, BATCH), jnp.int32),
        pltpu.VMEM((BATCH, DH), jnp.float32),
        pltpu.VMEM((BATCH, DH), jnp.float32),
    ]
    if with_count:
        out_type.append(jax.ShapeDtypeStruct((NP, 16), jnp.float32))
        scratch.append(pltpu.VMEM((BATCH, 16), jnp.float32))
    scratch.append(pltpu.VMEM_SHARED((NP, DH), jnp.float32))
    if with_count:
        scratch.append(pltpu.VMEM_SHARED((NP, 16), jnp.float32))
    scratch.append(pltpu.SemaphoreType.DMA)
    scratch.append(pltpu.SemaphoreType.DMA)
    return pl.kernel(
        functools.partial(_sc_agg_body, with_count),
        out_type=out_type,
        mesh=mesh,
        scratch_types=scratch,
        compiler_params=pltpu.CompilerParams(use_tc_tiling_on_sc=False),
    )


def _tc_layer_body(relu, acc_ref, cnt_ref, x_ref, wl_ref, wr_ref, b_ref, o_ref):
    agg = jnp.concatenate([acc_ref[0, :N, :], acc_ref[1, :N, :]], axis=1)
    if x_ref.shape[0] == NC:
        xin = jnp.concatenate([x_ref[0], x_ref[1]], axis=1)
    else:
        xin = x_ref[...]
    cnt = cnt_ref[:N, 0:1]
    mean = agg / jnp.maximum(cnt, 1.0)
    r = (jnp.dot(mean, wl_ref[...], preferred_element_type=jnp.float32)
         + jnp.dot(xin, wr_ref[...], preferred_element_type=jnp.float32)
         + b_ref[...])
    r = jnp.maximum(r, 0.0) if relu else r
    if o_ref.shape[0] == NC:
        o_ref[0] = r[:, :DH]
        o_ref[1] = r[:, DH:]
    else:
        o_ref[...] = r


def _make_tc_layer(relu, split_in, split_out):
    out_shape = (jax.ShapeDtypeStruct((NC, N, DH), jnp.float32) if split_out
                 else jax.ShapeDtypeStruct((N, D), jnp.float32))
    return pl.pallas_call(
        functools.partial(_tc_layer_body, relu),
        out_shape=out_shape,
    )


_sc_agg_cnt = _make_sc_agg(True)
_sc_agg = _make_sc_agg(False)
_tc_layer1 = _make_tc_layer(True, False, True)
_tc_layer2 = _make_tc_layer(False, True, False)


def kernel(x, edge_index, W1_l, W1_r, b1, W2_l, W2_r, b2):
    src = edge_index[0].astype(jnp.int32)
    dst = edge_index[1].astype(jnp.int32)
    pad = EP - E
    src_p = jnp.concatenate([src, jnp.zeros((pad,), jnp.int32)])
    dst_p = jnp.concatenate(
        [dst, N + (jnp.arange(pad, dtype=jnp.int32) % (NP - N))])
    src_p = src_p.reshape(EP // BATCH, BATCH)
    dst_p = dst_p.reshape(EP // BATCH, BATCH)
    x2 = x.reshape(N, NC, DH).transpose(1, 0, 2)

    acc1, cnt1 = _sc_agg_cnt(x2, src_p, dst_p)
    h2 = _tc_layer1(acc1, cnt1, x, W1_l, W1_r, b1.reshape(1, D))
    (acc2,) = _sc_agg(h2, src_p, dst_p)
    out = _tc_layer2(acc2, cnt1, h2, W2_l, W2_r, b2.reshape(1, D))
    return out
